# baseline (speedup 1.0000x reference)
import jax
import jax.numpy as jnp
from jax import lax
from jax.experimental import pallas as pl
from jax.experimental.pallas import tpu as pltpu
from jax.experimental.pallas import tpu_sc as plsc

N_CLS = 1000
D = 512
B = 16384

NC = 2
NS = 16
NH = NS // 2
L = 16
TW = 128
RPS = B // NH
CHUNK = 128
NCHUNK = RPS // CHUNK
NBUF = 6
CPAD = 1024
CPT = CPAD // NS
REAL_LAST = N_CLS - (NS - 1) * CPT


def _sc_body(audio, label, text, out,
             buf0, buf1, buf2, buf3, buf4, buf5,
             labels_v, cnt1_v, cnt16_v, recip_v,
             acc_sh, cnt_sh, semG, semS, semC):
    c = lax.axis_index("c")
    s = lax.axis_index("s")
    srow = lax.rem(s, NH)
    row0 = srow * RPS
    col = c * (2 * TW) + jnp.where(s < NH, 0, TW)
    off16 = jnp.full((L,), jnp.where(s < NH, 0, CPAD), jnp.int32)
    zeros16 = jnp.zeros((L,), jnp.float32)
    ones16 = jnp.ones((L,), jnp.float32)

    bufs = (buf0, buf1, buf2, buf3, buf4, buf5)
    gd = [None] * NCHUNK
    sd = [None] * NCHUNK

    def start_gather(j):
        gd[j] = pltpu.async_copy(
            audio.at[pl.ds(row0 + j * CHUNK, CHUNK), pl.ds(col, TW)],
            bufs[j % NBUF], semG)

    for j in range(4):
        start_gather(j)

    def zero_row(r, _):
        for v in range(TW // L):
            buf5[r, pl.ds(v * L, L)] = zeros16
        return 0
    lax.fori_loop(0, CHUNK, zero_row, 0)

    def zero_cnt(i, _):
        cnt1_v[pl.ds(i * L, L)] = zeros16
        return 0
    lax.fori_loop(0, CPAD // L, zero_cnt, 0)

    pltpu.sync_copy(buf5, acc_sh.at[pl.ds(s * CHUNK, CHUNK), :])

    def stage_labels(j, _):
        pltpu.sync_copy(label.at[pl.ds(row0 + j * CHUNK, CHUNK)], labels_v.at[j])
        return 0
    lax.fori_loop(0, NCHUNK, stage_labels, 0)

    def add_off(j, _):
        def add_vec(v, _):
            sl = pl.ds(v * L, L)
            labels_v[j, sl] = labels_v[j, sl] + off16
            return 0
        lax.fori_loop(0, CHUNK // L, add_vec, 0)
        return 0
    lax.fori_loop(0, NCHUNK, add_off, 0)

    @pl.when(s < NH)
    def _():
        def count_group(j, _):
            def count_vec(v, _):
                lbl = labels_v[j, pl.ds(v * L, L)]
                plsc.addupdate_scatter(cnt1_v, [lbl], ones16)
                return 0
            lax.fori_loop(0, CHUNK // L, count_vec, 0)
            return 0
        lax.fori_loop(0, NCHUNK, count_group, 0)

        def cnt_send(t, _):
            pltpu.async_copy(
                cnt1_v.at[pl.ds(t * CPT, CPT)],
                cnt_sh.at[pl.ds(t * (NH * CPT) + srow * CPT, CPT)], semC)
            return 0
        lax.fori_loop(0, NS, cnt_send, 0)

        def cnt_drain(t, _):
            pltpu.make_async_copy(
                cnt1_v.at[pl.ds(t * CPT, CPT)],
                cnt_sh.at[pl.ds(t * (NH * CPT) + srow * CPT, CPT)], semC).wait()
            return 0
        lax.fori_loop(0, NS, cnt_drain, 0)

    plsc.subcore_barrier()

    waited = [False] * NCHUNK
    for j in range(NCHUNK):
        if j + 4 < NCHUNK:
            if j >= 2:
                sd[j - 2].wait()
                waited[j - 2] = True
            start_gather(j + 4)
        gd[j].wait()
        sd[j] = pltpu.async_copy(
            bufs[j % NBUF], acc_sh.at[labels_v.at[j]], semS, add=True)
    for j in range(NCHUNK):
        if not waited[j]:
            sd[j].wait()

    plsc.subcore_barrier()

    k0 = s * CPT
    sums0_v = buf0
    sums1_v = buf1
    out0_v = buf2
    out1_v = buf3
    f0 = pltpu.async_copy(acc_sh.at[pl.ds(k0, CPT), :],
                          sums0_v.at[pl.ds(0, CPT), :], semG)
    f1 = pltpu.async_copy(acc_sh.at[pl.ds(CPAD + k0, CPT), :],
                          sums1_v.at[pl.ds(0, CPT), :], semG)
    f2 = pltpu.async_copy(cnt_sh.at[pl.ds(s * (NH * CPT), NH * CPT)],
                          cnt16_v, semC)

    colA = c * (2 * TW)
    colB = colA + TW

    @pl.when(s < NS - 1)
    def _():
        pltpu.sync_copy(text.at[pl.ds(k0, CPT), pl.ds(colA, TW)],
                        out0_v.at[pl.ds(0, CPT), :])
        pltpu.sync_copy(text.at[pl.ds(k0, CPT), pl.ds(colB, TW)],
                        out1_v.at[pl.ds(0, CPT), :])

    @pl.when(s == NS - 1)
    def _():
        rows = pl.ds((NS - 1) * CPT, REAL_LAST)
        pltpu.sync_copy(text.at[rows, pl.ds(colA, TW)],
                        out0_v.at[pl.ds(0, REAL_LAST), :])
        pltpu.sync_copy(text.at[rows, pl.ds(colB, TW)],
                        out1_v.at[pl.ds(0, REAL_LAST), :])

    f2.wait()

    def recip_vec(v, _):
        def accum(r, a):
            return a + cnt16_v[pl.ds(r * CPT + v * L, L)]
        a = lax.fori_loop(0, NH, accum, zeros16)
        recip_v[pl.ds(v * L, L)] = jnp.where(
            a > 0.0, 1.0 / jnp.maximum(a, 1.0), 0.0)
        return 0
    lax.fori_loop(0, CPT // L, recip_vec, 0)
    f0.wait()
    f1.wait()

    def out_row(r, _):
        ridx = jnp.full((L,), r, jnp.int32)
        rec = plsc.load_gather(recip_v, [ridx])
        for v in range(TW // L):
            sl = pl.ds(v * L, L)
            out0_v[r, sl] = out0_v[r, sl] + sums0_v[r, sl] * rec
            out1_v[r, sl] = out1_v[r, sl] + sums1_v[r, sl] * rec
        return 0
    lax.fori_loop(0, CPT, out_row, 0)

    @pl.when(s < NS - 1)
    def _():
        pltpu.sync_copy(out0_v.at[pl.ds(0, CPT), :],
                        out.at[pl.ds(k0, CPT), pl.ds(colA, TW)])
        pltpu.sync_copy(out1_v.at[pl.ds(0, CPT), :],
                        out.at[pl.ds(k0, CPT), pl.ds(colB, TW)])

    @pl.when(s == NS - 1)
    def _():
        rows = pl.ds((NS - 1) * CPT, REAL_LAST)
        pltpu.sync_copy(out0_v.at[pl.ds(0, REAL_LAST), :],
                        out.at[rows, pl.ds(colA, TW)])
        pltpu.sync_copy(out1_v.at[pl.ds(0, REAL_LAST), :],
                        out.at[rows, pl.ds(colB, TW)])


@jax.jit
def kernel(audio, label, text_proto):
    mesh = plsc.VectorSubcoreMesh(core_axis_name="c", subcore_axis_name="s")
    run = pl.kernel(
        _sc_body,
        out_type=jax.ShapeDtypeStruct((N_CLS, D), jnp.float32),
        mesh=mesh,
        scratch_types=[
            pltpu.VMEM((CHUNK, TW), jnp.float32),
            pltpu.VMEM((CHUNK, TW), jnp.float32),
            pltpu.VMEM((CHUNK, TW), jnp.float32),
            pltpu.VMEM((CHUNK, TW), jnp.float32),
            pltpu.VMEM((CHUNK, TW), jnp.float32),
            pltpu.VMEM((CHUNK, TW), jnp.float32),
            pltpu.VMEM((NCHUNK, CHUNK), jnp.int32),
            pltpu.VMEM((CPAD,), jnp.float32),
            pltpu.VMEM((NH * CPT,), jnp.float32),
            pltpu.VMEM((CPT,), jnp.float32),
            pltpu.VMEM_SHARED((2 * CPAD, TW), jnp.float32),
            pltpu.VMEM_SHARED((NS * NH * CPT,), jnp.float32),
            pltpu.SemaphoreType.DMA,
            pltpu.SemaphoreType.DMA,
            pltpu.SemaphoreType.DMA,
        ],
        compiler_params=pltpu.CompilerParams(
            use_tc_tiling_on_sc=True, needs_layout_passes=False),
        name="proto_evolution_sc",
    )
    return run(audio, label, text_proto)


# speedup vs baseline: 1.0003x; 1.0003x over previous
import jax
import jax.numpy as jnp
from jax import lax
from jax.experimental import pallas as pl
from jax.experimental.pallas import tpu as pltpu
from jax.experimental.pallas import tpu_sc as plsc

N_CLS = 1000
D = 512
B = 16384

NC = 2
NS = 16
L = 16
TW = 128
RPT = B // NS
CHUNK = 64
NCHUNK = RPT // CHUNK
NBUF = 6
CPAD = 1024
CPT = CPAD // NS
REAL_LAST = N_CLS - (NS - 1) * CPT


def _sc_body(audio, label, text, out,
             bufA0, bufA1, bufA2, bufA3, bufA4, bufA5,
             bufB0, bufB1, bufB2, bufB3, bufB4, bufB5,
             labels_v, cnt1_v, cnt16_v, recip_v,
             acc0_sh, acc1_sh, cnt_sh, semA, semB, semS, semC):
    c = lax.axis_index("c")
    s = lax.axis_index("s")
    row0 = s * RPT
    colA = c * (2 * TW)
    colB = colA + TW
    zeros16 = jnp.zeros((L,), jnp.float32)
    ones16 = jnp.ones((L,), jnp.float32)

    bufsA = (bufA0, bufA1, bufA2, bufA3, bufA4, bufA5)
    bufsB = (bufB0, bufB1, bufB2, bufB3, bufB4, bufB5)
    gA = [None] * NCHUNK
    gB = [None] * NCHUNK
    sA = [None] * NCHUNK
    sB = [None] * NCHUNK

    def start_gather(j):
        rows = pl.ds(row0 + j * CHUNK, CHUNK)
        gA[j] = pltpu.async_copy(
            audio.at[rows, pl.ds(colA, TW)], bufsA[j % NBUF], semA)
        gB[j] = pltpu.async_copy(
            audio.at[rows, pl.ds(colB, TW)], bufsB[j % NBUF], semB)

    for j in range(4):
        start_gather(j)

    def zero_row(r, _):
        for v in range(TW // L):
            bufB5[r, pl.ds(v * L, L)] = zeros16
        return 0
    lax.fori_loop(0, CHUNK, zero_row, 0)

    def zero_cnt(i, _):
        cnt1_v[pl.ds(i * L, L)] = zeros16
        return 0
    lax.fori_loop(0, CPAD // L, zero_cnt, 0)

    pltpu.sync_copy(bufB5, acc0_sh.at[pl.ds(s * CPT, CPT), :])
    pltpu.sync_copy(bufB5, acc1_sh.at[pl.ds(s * CPT, CPT), :])

    def stage_labels(j, _):
        pltpu.sync_copy(label.at[pl.ds(row0 + j * CHUNK, CHUNK)], labels_v.at[j])
        return 0
    lax.fori_loop(0, NCHUNK, stage_labels, 0)

    def count_group(j, _):
        def count_vec(v, _):
            lbl = labels_v[j, pl.ds(v * L, L)]
            plsc.addupdate_scatter(cnt1_v, [lbl], ones16)
            return 0
        lax.fori_loop(0, CHUNK // L, count_vec, 0)
        return 0
    lax.fori_loop(0, NCHUNK, count_group, 0)

    def cnt_send(t, _):
        pltpu.async_copy(
            cnt1_v.at[pl.ds(t * CPT, CPT)],
            cnt_sh.at[pl.ds(t * CPAD + s * CPT, CPT)], semC)
        return 0
    lax.fori_loop(0, NS, cnt_send, 0)

    def cnt_drain(t, _):
        pltpu.make_async_copy(
            cnt1_v.at[pl.ds(t * CPT, CPT)],
            cnt_sh.at[pl.ds(t * CPAD + s * CPT, CPT)], semC).wait()
        return 0
    lax.fori_loop(0, NS, cnt_drain, 0)

    plsc.subcore_barrier()

    waited = [False] * NCHUNK
    for j in range(NCHUNK):
        if j + 4 < NCHUNK:
            if j >= 2:
                sA[j - 2].wait()
                sB[j - 2].wait()
                waited[j - 2] = True
            start_gather(j + 4)
        gA[j].wait()
        gB[j].wait()
        idx = labels_v.at[j]
        sA[j] = pltpu.async_copy(bufsA[j % NBUF], acc0_sh.at[idx], semS, add=True)
        sB[j] = pltpu.async_copy(bufsB[j % NBUF], acc1_sh.at[idx], semS, add=True)
    for j in range(NCHUNK):
        if not waited[j]:
            sA[j].wait()
            sB[j].wait()

    plsc.subcore_barrier()

    k0 = s * CPT
    sums0_v = bufA0
    sums1_v = bufA1
    out0_v = bufB0
    out1_v = bufB1
    f0 = pltpu.async_copy(acc0_sh.at[pl.ds(k0, CPT), :], sums0_v, semA)
    f1 = pltpu.async_copy(acc1_sh.at[pl.ds(k0, CPT), :], sums1_v, semB)
    f2 = pltpu.async_copy(cnt_sh.at[pl.ds(s * CPAD, CPAD)], cnt16_v, semC)

    @pl.when(s < NS - 1)
    def _():
        pltpu.sync_copy(text.at[pl.ds(k0, CPT), pl.ds(colA, TW)], out0_v)
        pltpu.sync_copy(text.at[pl.ds(k0, CPT), pl.ds(colB, TW)], out1_v)

    @pl.when(s == NS - 1)
    def _():
        rows = pl.ds((NS - 1) * CPT, REAL_LAST)
        pltpu.sync_copy(text.at[rows, pl.ds(colA, TW)],
                        out0_v.at[pl.ds(0, REAL_LAST), :])
        pltpu.sync_copy(text.at[rows, pl.ds(colB, TW)],
                        out1_v.at[pl.ds(0, REAL_LAST), :])

    f2.wait()

    def recip_vec(v, _):
        def accum(r, a):
            return a + cnt16_v[pl.ds(r * CPT + v * L, L)]
        a = lax.fori_loop(0, NS, accum, zeros16)
        recip_v[pl.ds(v * L, L)] = jnp.where(
            a > 0.0, 1.0 / jnp.maximum(a, 1.0), 0.0)
        return 0
    lax.fori_loop(0, CPT // L, recip_vec, 0)
    f0.wait()
    f1.wait()

    def out_row(r, _):
        ridx = jnp.full((L,), r, jnp.int32)
        rec = plsc.load_gather(recip_v, [ridx])
        for v in range(TW // L):
            sl = pl.ds(v * L, L)
            out0_v[r, sl] = out0_v[r, sl] + sums0_v[r, sl] * rec
            out1_v[r, sl] = out1_v[r, sl] + sums1_v[r, sl] * rec
        return 0
    lax.fori_loop(0, CPT, out_row, 0)

    @pl.when(s < NS - 1)
    def _():
        pltpu.sync_copy(out0_v, out.at[pl.ds(k0, CPT), pl.ds(colA, TW)])
        pltpu.sync_copy(out1_v, out.at[pl.ds(k0, CPT), pl.ds(colB, TW)])

    @pl.when(s == NS - 1)
    def _():
        rows = pl.ds((NS - 1) * CPT, REAL_LAST)
        pltpu.sync_copy(out0_v.at[pl.ds(0, REAL_LAST), :],
                        out.at[rows, pl.ds(colA, TW)])
        pltpu.sync_copy(out1_v.at[pl.ds(0, REAL_LAST), :],
                        out.at[rows, pl.ds(colB, TW)])


@jax.jit
def kernel(audio, label, text_proto):
    mesh = plsc.VectorSubcoreMesh(core_axis_name="c", subcore_axis_name="s")
    run = pl.kernel(
        _sc_body,
        out_type=jax.ShapeDtypeStruct((N_CLS, D), jnp.float32),
        mesh=mesh,
        scratch_types=[
            pltpu.VMEM((CHUNK, TW), jnp.float32),
            pltpu.VMEM((CHUNK, TW), jnp.float32),
            pltpu.VMEM((CHUNK, TW), jnp.float32),
            pltpu.VMEM((CHUNK, TW), jnp.float32),
            pltpu.VMEM((CHUNK, TW), jnp.float32),
            pltpu.VMEM((CHUNK, TW), jnp.float32),
            pltpu.VMEM((CHUNK, TW), jnp.float32),
            pltpu.VMEM((CHUNK, TW), jnp.float32),
            pltpu.VMEM((CHUNK, TW), jnp.float32),
            pltpu.VMEM((CHUNK, TW), jnp.float32),
            pltpu.VMEM((CHUNK, TW), jnp.float32),
            pltpu.VMEM((CHUNK, TW), jnp.float32),
            pltpu.VMEM((NCHUNK, CHUNK), jnp.int32),
            pltpu.VMEM((CPAD,), jnp.float32),
            pltpu.VMEM((CPAD,), jnp.float32),
            pltpu.VMEM((CPT,), jnp.float32),
            pltpu.VMEM_SHARED((CPAD, TW), jnp.float32),
            pltpu.VMEM_SHARED((CPAD, TW), jnp.float32),
            pltpu.VMEM_SHARED((NS * CPAD,), jnp.float32),
            pltpu.SemaphoreType.DMA,
            pltpu.SemaphoreType.DMA,
            pltpu.SemaphoreType.DMA,
            pltpu.SemaphoreType.DMA,
        ],
        compiler_params=pltpu.CompilerParams(
            use_tc_tiling_on_sc=True, needs_layout_passes=False),
        name="proto_evolution_sc",
    )
    return run(audio, label, text_proto)


# speedup vs baseline: 1.0646x; 1.0643x over previous
import jax
import jax.numpy as jnp
from jax import lax
from jax.experimental import pallas as pl
from jax.experimental.pallas import tpu as pltpu
from jax.experimental.pallas import tpu_sc as plsc

N_CLS = 1000
D = 512
B = 16384

NC = 2
NS = 16
L = 16
TW = 128
RPT = B // NS
CHUNK = 128
NCHUNK = RPT // CHUNK
NBUF = 3
CPAD = 1024
CPT = CPAD // NS
REAL_LAST = N_CLS - (NS - 1) * CPT


def _sc_body(audio, label, text, out,
             bufA0, bufA1, bufA2, bufB0, bufB1, bufB2,
             labels_v, cnt1_v, cnt16_v, recip_v,
             acc0_sh, acc1_sh, cnt_sh, semA, semB, semS, semC):
    c = lax.axis_index("c")
    s = lax.axis_index("s")
    row0 = s * RPT
    colA = c * (2 * TW)
    colB = colA + TW
    zeros16 = jnp.zeros((L,), jnp.float32)
    ones16 = jnp.ones((L,), jnp.float32)

    bufsA = (bufA0, bufA1, bufA2)
    bufsB = (bufB0, bufB1, bufB2)
    gA = [None] * NCHUNK
    gB = [None] * NCHUNK
    sA = [None] * NCHUNK
    sB = [None] * NCHUNK

    def start_gather(j):
        rows = pl.ds(row0 + j * CHUNK, CHUNK)
        gA[j] = pltpu.async_copy(
            audio.at[rows, pl.ds(colA, TW)], bufsA[j % NBUF], semA)
        gB[j] = pltpu.async_copy(
            audio.at[rows, pl.ds(colB, TW)], bufsB[j % NBUF], semB)

    start_gather(0)
    start_gather(1)

    def zero_row(r, _):
        for v in range(TW // L):
            bufB2[r, pl.ds(v * L, L)] = zeros16
        return 0
    lax.fori_loop(0, CPT, zero_row, 0)

    def zero_cnt(i, _):
        cnt1_v[pl.ds(i * L, L)] = zeros16
        return 0
    lax.fori_loop(0, CPAD // L, zero_cnt, 0)

    zsrc = bufB2.at[pl.ds(0, CPT), :]
    pltpu.sync_copy(zsrc, acc0_sh.at[pl.ds(s * CPT, CPT), :])
    pltpu.sync_copy(zsrc, acc1_sh.at[pl.ds(s * CPT, CPT), :])

    def stage_labels(j, _):
        pltpu.sync_copy(label.at[pl.ds(row0 + j * CHUNK, CHUNK)], labels_v.at[j])
        return 0
    lax.fori_loop(0, NCHUNK, stage_labels, 0)

    def count_group(j, _):
        def count_vec(v, _):
            lbl = labels_v[j, pl.ds(v * L, L)]
            plsc.addupdate_scatter(cnt1_v, [lbl], ones16)
            return 0
        lax.fori_loop(0, CHUNK // L, count_vec, 0)
        return 0
    lax.fori_loop(0, NCHUNK, count_group, 0)

    def cnt_send(t, _):
        pltpu.async_copy(
            cnt1_v.at[pl.ds(t * CPT, CPT)],
            cnt_sh.at[pl.ds(t * CPAD + s * CPT, CPT)], semC)
        return 0
    lax.fori_loop(0, NS, cnt_send, 0)

    def cnt_drain(t, _):
        pltpu.make_async_copy(
            cnt1_v.at[pl.ds(t * CPT, CPT)],
            cnt_sh.at[pl.ds(t * CPAD + s * CPT, CPT)], semC).wait()
        return 0
    lax.fori_loop(0, NS, cnt_drain, 0)

    plsc.subcore_barrier()

    waited = [False] * NCHUNK
    for j in range(NCHUNK):
        if j + 2 < NCHUNK:
            if j >= 1:
                sA[j - 1].wait()
                sB[j - 1].wait()
                waited[j - 1] = True
            start_gather(j + 2)
        gA[j].wait()
        gB[j].wait()
        idx = labels_v.at[j]
        sA[j] = pltpu.async_copy(bufsA[j % NBUF], acc0_sh.at[idx], semS, add=True)
        sB[j] = pltpu.async_copy(bufsB[j % NBUF], acc1_sh.at[idx], semS, add=True)
    for j in range(NCHUNK):
        if not waited[j]:
            sA[j].wait()
            sB[j].wait()

    plsc.subcore_barrier()

    k0 = s * CPT
    sums0_v = bufA0
    sums1_v = bufA1
    out0_v = bufB0
    out1_v = bufB1
    f0 = pltpu.async_copy(acc0_sh.at[pl.ds(k0, CPT), :],
                          sums0_v.at[pl.ds(0, CPT), :], semA)
    f1 = pltpu.async_copy(acc1_sh.at[pl.ds(k0, CPT), :],
                          sums1_v.at[pl.ds(0, CPT), :], semB)
    f2 = pltpu.async_copy(cnt_sh.at[pl.ds(s * CPAD, CPAD)], cnt16_v, semC)

    @pl.when(s < NS - 1)
    def _():
        pltpu.sync_copy(text.at[pl.ds(k0, CPT), pl.ds(colA, TW)],
                        out0_v.at[pl.ds(0, CPT), :])
        pltpu.sync_copy(text.at[pl.ds(k0, CPT), pl.ds(colB, TW)],
                        out1_v.at[pl.ds(0, CPT), :])

    @pl.when(s == NS - 1)
    def _():
        rows = pl.ds((NS - 1) * CPT, REAL_LAST)
        pltpu.sync_copy(text.at[rows, pl.ds(colA, TW)],
                        out0_v.at[pl.ds(0, REAL_LAST), :])
        pltpu.sync_copy(text.at[rows, pl.ds(colB, TW)],
                        out1_v.at[pl.ds(0, REAL_LAST), :])

    f2.wait()

    def recip_vec(v, _):
        def accum(r, a):
            return a + cnt16_v[pl.ds(r * CPT + v * L, L)]
        a = lax.fori_loop(0, NS, accum, zeros16)
        recip_v[pl.ds(v * L, L)] = jnp.where(
            a > 0.0, 1.0 / jnp.maximum(a, 1.0), 0.0)
        return 0
    lax.fori_loop(0, CPT // L, recip_vec, 0)
    f0.wait()
    f1.wait()

    def out_row(r, _):
        ridx = jnp.full((L,), r, jnp.int32)
        rec = plsc.load_gather(recip_v, [ridx])
        for v in range(TW // L):
            sl = pl.ds(v * L, L)
            out0_v[r, sl] = out0_v[r, sl] + sums0_v[r, sl] * rec
            out1_v[r, sl] = out1_v[r, sl] + sums1_v[r, sl] * rec
        return 0
    lax.fori_loop(0, CPT, out_row, 0)

    @pl.when(s < NS - 1)
    def _():
        pltpu.sync_copy(out0_v.at[pl.ds(0, CPT), :],
                        out.at[pl.ds(k0, CPT), pl.ds(colA, TW)])
        pltpu.sync_copy(out1_v.at[pl.ds(0, CPT), :],
                        out.at[pl.ds(k0, CPT), pl.ds(colB, TW)])

    @pl.when(s == NS - 1)
    def _():
        rows = pl.ds((NS - 1) * CPT, REAL_LAST)
        pltpu.sync_copy(out0_v.at[pl.ds(0, REAL_LAST), :],
                        out.at[rows, pl.ds(colA, TW)])
        pltpu.sync_copy(out1_v.at[pl.ds(0, REAL_LAST), :],
                        out.at[rows, pl.ds(colB, TW)])


@jax.jit
def kernel(audio, label, text_proto):
    mesh = plsc.VectorSubcoreMesh(core_axis_name="c", subcore_axis_name="s")
    run = pl.kernel(
        _sc_body,
        out_type=jax.ShapeDtypeStruct((N_CLS, D), jnp.float32),
        mesh=mesh,
        scratch_types=[
            pltpu.VMEM((CHUNK, TW), jnp.float32),
            pltpu.VMEM((CHUNK, TW), jnp.float32),
            pltpu.VMEM((CHUNK, TW), jnp.float32),
            pltpu.VMEM((CHUNK, TW), jnp.float32),
            pltpu.VMEM((CHUNK, TW), jnp.float32),
            pltpu.VMEM((CHUNK, TW), jnp.float32),
            pltpu.VMEM((NCHUNK, CHUNK), jnp.int32),
            pltpu.VMEM((CPAD,), jnp.float32),
            pltpu.VMEM((CPAD,), jnp.float32),
            pltpu.VMEM((CPT,), jnp.float32),
            pltpu.VMEM_SHARED((CPAD, TW), jnp.float32),
            pltpu.VMEM_SHARED((CPAD, TW), jnp.float32),
            pltpu.VMEM_SHARED((NS * CPAD,), jnp.float32),
            pltpu.SemaphoreType.DMA,
            pltpu.SemaphoreType.DMA,
            pltpu.SemaphoreType.DMA,
            pltpu.SemaphoreType.DMA,
        ],
        compiler_params=pltpu.CompilerParams(
            use_tc_tiling_on_sc=True, needs_layout_passes=False),
        name="proto_evolution_sc",
    )
    return run(audio, label, text_proto)
